# Initial kernel scaffold; baseline (speedup 1.0000x reference)
import functools

import jax
import jax.numpy as jnp
from jax import lax
from jax.experimental import pallas as pl
from jax.experimental.pallas import tpu as pltpu
from jax.experimental.pallas import tpu_sc as plsc

N = 10000
D = 128
E = 320000
EPS = 1e-5
NC, NS = 2, 16
NTILES = NC * NS
CHUNK = 128
CPT = 79
EPT = CPT * CHUNK
E_PAD = NTILES * EPT
NPAD = 10016
ROWS_PT = NPAD // NS
DEG_R = 10112
DEG_RPT = DEG_R // NS

_MESH = plsc.VectorSubcoreMesh(core_axis_name="c", subcore_axis_name="s")


def _deg_body(dst_hbm, out_hbm, dst_v, ones_v, zeros_v, deg_sh):
    c = lax.axis_index("c")
    s = lax.axis_index("s")
    tid = c * NS + s

    def fill_ones(j, carry):
        ones_v[j] = jnp.full((16,), 1.0, jnp.float32)
        return carry

    lax.fori_loop(0, CHUNK, fill_ones, 0)

    def fill_zeros(j, carry):
        zeros_v[j] = jnp.zeros((16,), jnp.float32)
        return carry

    lax.fori_loop(0, DEG_RPT, fill_zeros, 0)
    pltpu.sync_copy(zeros_v, deg_sh.at[pl.ds(s * DEG_RPT, DEG_RPT)])
    plsc.subcore_barrier()

    base = tid * EPT

    def step(j, carry):
        pltpu.sync_copy(dst_hbm.at[pl.ds(base + j * CHUNK, CHUNK)], dst_v)
        pltpu.sync_copy(ones_v, deg_sh.at[dst_v], add=True)
        return carry

    lax.fori_loop(0, CPT, step, 0)
    plsc.subcore_barrier()
    pltpu.sync_copy(
        deg_sh.at[pl.ds(s * DEG_RPT, DEG_RPT)],
        out_hbm.at[c, pl.ds(s * DEG_RPT, DEG_RPT)],
    )


_deg_call = pl.kernel(
    _deg_body,
    out_type=jax.ShapeDtypeStruct((NC, DEG_R, 16), jnp.float32),
    mesh=_MESH,
    scratch_types=[
        pltpu.VMEM((CHUNK,), jnp.int32),
        pltpu.VMEM((CHUNK, 16), jnp.float32),
        pltpu.VMEM((DEG_RPT, 16), jnp.float32),
        pltpu.VMEM_SHARED((DEG_R, 16), jnp.float32),
    ],
)


def _agg_body(hn_hbm, src_hbm, dst_hbm, out_hbm, src_v, dst_v, rows_v, agg_sh, sem):
    c = lax.axis_index("c")
    s = lax.axis_index("s")
    tid = c * NS + s

    pltpu.sync_copy(
        hn_hbm.at[pl.ds(s * ROWS_PT, ROWS_PT)],
        agg_sh.at[pl.ds(s * ROWS_PT, ROWS_PT)],
    )
    plsc.subcore_barrier()

    base = tid * EPT

    def step(j, carry):
        pltpu.sync_copy(src_hbm.at[pl.ds(base + j * CHUNK, CHUNK)], src_v)
        pltpu.sync_copy(dst_hbm.at[pl.ds(base + j * CHUNK, CHUNK)], dst_v)
        pltpu.async_copy(hn_hbm.at[src_v], rows_v, sem).wait()
        pltpu.sync_copy(rows_v, agg_sh.at[dst_v], add=True)
        return carry

    lax.fori_loop(0, CPT, step, 0)
    plsc.subcore_barrier()
    pltpu.sync_copy(
        agg_sh.at[pl.ds(s * ROWS_PT, ROWS_PT)],
        out_hbm.at[c, pl.ds(s * ROWS_PT, ROWS_PT)],
    )


_agg_call = pl.kernel(
    _agg_body,
    out_type=jax.ShapeDtypeStruct((NC, NPAD, D), jnp.float32),
    mesh=_MESH,
    scratch_types=[
        pltpu.VMEM((CHUNK,), jnp.int32),
        pltpu.VMEM((CHUNK,), jnp.int32),
        pltpu.VMEM((CHUNK, D), jnp.float32),
        pltpu.VMEM_SHARED((NPAD, D), jnp.float32),
        pltpu.SemaphoreType.DMA,
    ],
)


def _lin_body(x_ref, w_ref, d0_ref, d1_ref, hn_ref):
    deg = d0_ref[...] + d1_ref[...] + 1.0
    norm = lax.rsqrt(deg)
    h = lax.dot_general(
        x_ref[...], w_ref[...], (((1,), (1,)), ((), ())),
        preferred_element_type=jnp.float32,
    )
    hn_ref[...] = h * norm


ROWS_B = 1000

_lin_call = pl.pallas_call(
    _lin_body,
    grid=(N // ROWS_B,),
    in_specs=[
        pl.BlockSpec((ROWS_B, D), lambda i: (i, 0)),
        pl.BlockSpec((D, D), lambda i: (0, 0)),
        pl.BlockSpec((ROWS_B, 1), lambda i: (i, 0)),
        pl.BlockSpec((ROWS_B, 1), lambda i: (i, 0)),
    ],
    out_specs=pl.BlockSpec((ROWS_B, D), lambda i: (i, 0)),
    out_shape=jax.ShapeDtypeStruct((N, D), jnp.float32),
)


def _ln_body(p0_ref, p1_ref, hn_ref, x_ref, d0_ref, d1_ref, g_ref, b_ref, o_ref):
    deg = d0_ref[...] + d1_ref[...] + 1.0
    norm = lax.rsqrt(deg)
    agg = (p0_ref[...] + p1_ref[...] - hn_ref[...]) * norm
    h = agg + x_ref[...]
    mean = jnp.mean(h, axis=-1, keepdims=True)
    cent = h - mean
    var = jnp.mean(cent * cent, axis=-1, keepdims=True)
    hln = cent * lax.rsqrt(var + EPS) * g_ref[0:1, :] + b_ref[0:1, :]
    o_ref[...] = jnp.maximum(hln, 0.0)


_ln_call = pl.pallas_call(
    _ln_body,
    grid=(N // ROWS_B,),
    in_specs=[
        pl.BlockSpec((ROWS_B, D), lambda i: (i, 0)),
        pl.BlockSpec((ROWS_B, D), lambda i: (i, 0)),
        pl.BlockSpec((ROWS_B, D), lambda i: (i, 0)),
        pl.BlockSpec((ROWS_B, D), lambda i: (i, 0)),
        pl.BlockSpec((ROWS_B, 1), lambda i: (i, 0)),
        pl.BlockSpec((ROWS_B, 1), lambda i: (i, 0)),
        pl.BlockSpec((8, D), lambda i: (0, 0)),
        pl.BlockSpec((8, D), lambda i: (0, 0)),
    ],
    out_specs=pl.BlockSpec((ROWS_B, D), lambda i: (i, 0)),
    out_shape=jax.ShapeDtypeStruct((N, D), jnp.float32),
)


@jax.jit
def kernel(x, edge_index, W, ln_gamma, ln_beta):
    ei = edge_index.astype(jnp.int32)
    pad = jnp.full((E_PAD - E,), N, jnp.int32)
    src_p = jnp.concatenate([ei[0], pad])
    dst_p = jnp.concatenate([ei[1], pad])

    deg_parts = _deg_call(dst_p)
    d0 = deg_parts[0, :N, 0].reshape(N, 1)
    d1 = deg_parts[1, :N, 0].reshape(N, 1)

    hn = _lin_call(x, W, d0, d1)
    hn_pad = jnp.concatenate([hn, jnp.zeros((NPAD - N, D), jnp.float32)])

    parts = _agg_call(hn_pad, src_p, dst_p)

    g8 = jnp.broadcast_to(ln_gamma.reshape(1, D), (8, D))
    b8 = jnp.broadcast_to(ln_beta.reshape(1, D), (8, D))
    return _ln_call(parts[0, :N], parts[1, :N], hn, x, d0, d1, g8, b8)


# speedup vs baseline: 4.9928x; 4.9928x over previous
import functools

import jax
import jax.numpy as jnp
from jax import lax
from jax.experimental import pallas as pl
from jax.experimental.pallas import tpu as pltpu
from jax.experimental.pallas import tpu_sc as plsc

N = 10000
D = 128
E = 320000
EPS = 1e-5
NC, NS = 2, 16
NTILES = NC * NS
CHUNK = 128
CPT = 79
EPT = CPT * CHUNK
E_PAD = NTILES * EPT
NPAD = 10112
ROWS_PT = NPAD // NS

_MESH = plsc.VectorSubcoreMesh(core_axis_name="c", subcore_axis_name="s")


def _deg_body(dst_hbm, ones_hbm, out_hbm, dst_v, ones_v, zbuf_v, deg_sh):
    c = lax.axis_index("c")
    s = lax.axis_index("s")
    tid = c * NS + s

    pltpu.sync_copy(ones_hbm.at[pl.ds(0, CHUNK)], ones_v)

    def zloop(k, carry):
        off = s * ROWS_PT + k * CHUNK
        pltpu.sync_copy(zbuf_v, deg_sh.at[pl.ds(off, CHUNK)])
        return carry

    pltpu.sync_copy(ones_hbm.at[pl.ds(CHUNK, CHUNK)], zbuf_v)
    lax.fori_loop(0, ROWS_PT // CHUNK, zloop, 0)
    tail = ROWS_PT % CHUNK
    toff = s * ROWS_PT + (ROWS_PT // CHUNK) * CHUNK
    pltpu.sync_copy(zbuf_v.at[pl.ds(0, tail)], deg_sh.at[pl.ds(toff, tail)])
    plsc.subcore_barrier()

    base = tid * EPT

    def step(j, carry):
        pltpu.sync_copy(dst_hbm.at[pl.ds(base + j * CHUNK, CHUNK)], dst_v)
        pltpu.sync_copy(ones_v, deg_sh.at[dst_v], add=True)
        return carry

    lax.fori_loop(0, CPT, step, 0)
    plsc.subcore_barrier()

    def writeback(k, carry):
        off = s * ROWS_PT + k * CHUNK
        pltpu.sync_copy(deg_sh.at[pl.ds(off, CHUNK)], zbuf_v)
        pltpu.sync_copy(zbuf_v, out_hbm.at[pl.ds(c * NPAD + off, CHUNK)])
        return carry

    lax.fori_loop(0, ROWS_PT // CHUNK, writeback, 0)
    pltpu.sync_copy(deg_sh.at[pl.ds(toff, tail)], zbuf_v.at[pl.ds(0, tail)])
    pltpu.sync_copy(zbuf_v.at[pl.ds(0, tail)], out_hbm.at[pl.ds(c * NPAD + toff, tail)])


_deg_call = pl.kernel(
    _deg_body,
    out_type=jax.ShapeDtypeStruct((NC * NPAD, D), jnp.float32),
    mesh=_MESH,
    scratch_types=[
        pltpu.VMEM((CHUNK,), jnp.int32),
        pltpu.VMEM((CHUNK, D), jnp.float32),
        pltpu.VMEM((CHUNK, D), jnp.float32),
        pltpu.VMEM_SHARED((NPAD, D), jnp.float32),
    ],
)


def _agg_body(hn_hbm, src_hbm, dst_hbm, out_hbm, src_v, dst_v, rows_v,
              agg_sh, sem):
    c = lax.axis_index("c")
    s = lax.axis_index("s")
    tid = c * NS + s

    def stage(k, carry):
        off = s * ROWS_PT + k * CHUNK
        pltpu.sync_copy(hn_hbm.at[pl.ds(off, CHUNK)], rows_v)
        pltpu.sync_copy(rows_v, agg_sh.at[pl.ds(off, CHUNK)])
        return carry

    lax.fori_loop(0, ROWS_PT // CHUNK, stage, 0)
    tail = ROWS_PT % CHUNK
    toff = s * ROWS_PT + (ROWS_PT // CHUNK) * CHUNK
    pltpu.sync_copy(hn_hbm.at[pl.ds(toff, tail)], rows_v.at[pl.ds(0, tail)])
    pltpu.sync_copy(rows_v.at[pl.ds(0, tail)], agg_sh.at[pl.ds(toff, tail)])
    plsc.subcore_barrier()

    base = tid * EPT

    def step(j, carry):
        pltpu.sync_copy(src_hbm.at[pl.ds(base + j * CHUNK, CHUNK)], src_v)
        pltpu.sync_copy(dst_hbm.at[pl.ds(base + j * CHUNK, CHUNK)], dst_v)
        pltpu.async_copy(hn_hbm.at[src_v], rows_v, sem).wait()
        pltpu.sync_copy(rows_v, agg_sh.at[dst_v], add=True)
        return carry

    lax.fori_loop(0, CPT, step, 0)
    plsc.subcore_barrier()

    def writeback(k, carry):
        off = s * ROWS_PT + k * CHUNK
        pltpu.sync_copy(agg_sh.at[pl.ds(off, CHUNK)], rows_v)
        pltpu.sync_copy(rows_v, out_hbm.at[pl.ds(c * NPAD + off, CHUNK)])
        return carry

    lax.fori_loop(0, ROWS_PT // CHUNK, writeback, 0)
    pltpu.sync_copy(agg_sh.at[pl.ds(toff, tail)], rows_v.at[pl.ds(0, tail)])
    pltpu.sync_copy(rows_v.at[pl.ds(0, tail)], out_hbm.at[pl.ds(c * NPAD + toff, tail)])


_agg_call = pl.kernel(
    _agg_body,
    out_type=jax.ShapeDtypeStruct((NC * NPAD, D), jnp.float32),
    mesh=_MESH,
    scratch_types=[
        pltpu.VMEM((CHUNK,), jnp.int32),
        pltpu.VMEM((CHUNK,), jnp.int32),
        pltpu.VMEM((CHUNK, D), jnp.float32),
        pltpu.VMEM_SHARED((NPAD, D), jnp.float32),
        pltpu.SemaphoreType.DMA,
    ],
)


def _lin_body(x_ref, w_ref, d0_ref, d1_ref, hn_ref):
    deg = d0_ref[...] + d1_ref[...] + 1.0
    norm = lax.rsqrt(deg)
    h = lax.dot_general(
        x_ref[...], w_ref[...], (((1,), (1,)), ((), ())),
        preferred_element_type=jnp.float32,
    )
    hn_ref[...] = h * norm


ROWS_B = 1000

_lin_call = pl.pallas_call(
    _lin_body,
    grid=(N // ROWS_B,),
    in_specs=[
        pl.BlockSpec((ROWS_B, D), lambda i: (i, 0)),
        pl.BlockSpec((D, D), lambda i: (0, 0)),
        pl.BlockSpec((ROWS_B, 1), lambda i: (i, 0)),
        pl.BlockSpec((ROWS_B, 1), lambda i: (i, 0)),
    ],
    out_specs=pl.BlockSpec((ROWS_B, D), lambda i: (i, 0)),
    out_shape=jax.ShapeDtypeStruct((N, D), jnp.float32),
)


def _ln_body(p0_ref, p1_ref, hn_ref, x_ref, d0_ref, d1_ref, g_ref, b_ref, o_ref):
    deg = d0_ref[...] + d1_ref[...] + 1.0
    norm = lax.rsqrt(deg)
    agg = (p0_ref[...] + p1_ref[...] - hn_ref[...]) * norm
    h = agg + x_ref[...]
    mean = jnp.mean(h, axis=-1, keepdims=True)
    cent = h - mean
    var = jnp.mean(cent * cent, axis=-1, keepdims=True)
    hln = cent * lax.rsqrt(var + EPS) * g_ref[0:1, :] + b_ref[0:1, :]
    o_ref[...] = jnp.maximum(hln, 0.0)


_ln_call = pl.pallas_call(
    _ln_body,
    grid=(N // ROWS_B,),
    in_specs=[
        pl.BlockSpec((ROWS_B, D), lambda i: (i, 0)),
        pl.BlockSpec((ROWS_B, D), lambda i: (i, 0)),
        pl.BlockSpec((ROWS_B, D), lambda i: (i, 0)),
        pl.BlockSpec((ROWS_B, D), lambda i: (i, 0)),
        pl.BlockSpec((ROWS_B, 1), lambda i: (i, 0)),
        pl.BlockSpec((ROWS_B, 1), lambda i: (i, 0)),
        pl.BlockSpec((8, D), lambda i: (0, 0)),
        pl.BlockSpec((8, D), lambda i: (0, 0)),
    ],
    out_specs=pl.BlockSpec((ROWS_B, D), lambda i: (i, 0)),
    out_shape=jax.ShapeDtypeStruct((N, D), jnp.float32),
)


@jax.jit
def kernel(x, edge_index, W, ln_gamma, ln_beta):
    ei = edge_index.astype(jnp.int32)
    pad = jnp.full((E_PAD - E,), N, jnp.int32)
    src_p = jnp.concatenate([ei[0], pad])
    dst_p = jnp.concatenate([ei[1], pad])

    ones_c = jnp.concatenate([
        jnp.ones((CHUNK, D), jnp.float32),
        jnp.zeros((CHUNK, D), jnp.float32),
    ])
    deg_parts = _deg_call(dst_p, ones_c)
    d0 = deg_parts[:N, 0:1]
    d1 = deg_parts[NPAD:NPAD + N, 0:1]

    hn = _lin_call(x, W, d0, d1)
    hn_pad = jnp.concatenate([hn, jnp.zeros((NPAD - N, D), jnp.float32)])

    parts = _agg_call(hn_pad, src_p, dst_p)

    g8 = jnp.broadcast_to(ln_gamma.reshape(1, D), (8, D))
    b8 = jnp.broadcast_to(ln_beta.reshape(1, D), (8, D))
    return _ln_call(parts[:N], parts[NPAD:NPAD + N], hn, x, d0, d1, g8, b8)
